# Initial kernel scaffold; baseline (speedup 1.0000x reference)
import jax
import jax.numpy as jnp
from jax.experimental import pallas as pl


def kernel(x, edge_index, edge_attr, pos, batch, params):
    raise NotImplementedError("write your pallas kernel here")


# speedup vs baseline: 2.9654x; 2.9654x over previous
import functools

import jax
import jax.numpy as jnp
from jax import lax
from jax.experimental import pallas as pl
from jax.experimental.pallas import tpu as pltpu
from jax.experimental.pallas import tpu_sc as plsc

N = 10000
E = 320000
D = 128
ED = 16
G = 16

NC = 2
NS = 16
NW = NC * NS
EW = E // NW
CH = 80
NCHUNK = EW // CH
NP = 10240
RPT = NP // NS
RZ = 128

BN = 1000
GN = N // BN
BE = 4000
GE = E // BE


def _mm(a, w):
    return lax.dot_general(a, w, (((1,), (0,)), ((), ())),
                           preferred_element_type=jnp.float32,
                           precision=lax.Precision.HIGHEST)


def _emb_body(x_ref, w_ref, b_ref, o_ref):
    o_ref[...] = _mm(x_ref[...], w_ref[...]) + b_ref[...]


_emb_call = pl.pallas_call(
    _emb_body,
    grid=(GN,),
    in_specs=[
        pl.BlockSpec((BN, D), lambda i: (i, 0)),
        pl.BlockSpec((D, D), lambda i: (0, 0)),
        pl.BlockSpec((1, D), lambda i: (0, 0)),
    ],
    out_specs=pl.BlockSpec((BN, D), lambda i: (i, 0)),
    out_shape=jax.ShapeDtypeStruct((N, D), jnp.float32),
)


def _stage1_body(h_ref, w1_ref, b1_ref, w2_ref, b2_ref, ea_ref, eb_ref,
                 hn_ref, a_ref, b_ref):
    h = h_ref[...]
    p = jnp.maximum(_mm(h, w1_ref[...]) + b1_ref[...], 0.0)
    hn_ref[...] = _mm(p, w2_ref[...]) + b2_ref[...]
    a_ref[...] = _mm(h, ea_ref[...])
    b_ref[...] = _mm(h, eb_ref[...])


_stage1_call = pl.pallas_call(
    _stage1_body,
    grid=(GN,),
    in_specs=[
        pl.BlockSpec((BN, D), lambda i: (i, 0)),
        pl.BlockSpec((D, D), lambda i: (0, 0)),
        pl.BlockSpec((1, D), lambda i: (0, 0)),
        pl.BlockSpec((D, D), lambda i: (0, 0)),
        pl.BlockSpec((1, D), lambda i: (0, 0)),
        pl.BlockSpec((D, D), lambda i: (0, 0)),
        pl.BlockSpec((D, D), lambda i: (0, 0)),
    ],
    out_specs=[pl.BlockSpec((BN, D), lambda i: (i, 0))] * 3,
    out_shape=[jax.ShapeDtypeStruct((N, D), jnp.float32)] * 3,
)


def _cmat_body(ea_ref, w_ref, b_ref, o_ref):
    o_ref[...] = _mm(ea_ref[...], w_ref[...]) + b_ref[...]


_cmat_call = pl.pallas_call(
    _cmat_body,
    grid=(GE,),
    in_specs=[
        pl.BlockSpec((BE, ED), lambda i: (i, 0)),
        pl.BlockSpec((ED, D), lambda i: (0, 0)),
        pl.BlockSpec((1, D), lambda i: (0, 0)),
    ],
    out_specs=pl.BlockSpec((BE, D), lambda i: (i, 0)),
    out_shape=jax.ShapeDtypeStruct((E, D), jnp.float32),
)


def _stage2_body(hn_ref, s_ref, deg_ref, e2w_ref, e2b_ref, uwa_ref, uwb_ref,
                 ub_ref, o_ref):
    s = s_ref[0] + s_ref[1]
    degc = deg_ref[0, :, 0:1] + deg_ref[1, :, 0:1]
    hagg = _mm(s, e2w_ref[...]) + degc * e2b_ref[...]
    o_ref[...] = _mm(hn_ref[...], uwa_ref[...]) + _mm(hagg, uwb_ref[...]) + ub_ref[...]


_stage2_call = pl.pallas_call(
    _stage2_body,
    grid=(GN,),
    in_specs=[
        pl.BlockSpec((BN, D), lambda i: (i, 0)),
        pl.BlockSpec((NC, BN, D), lambda i: (0, i, 0)),
        pl.BlockSpec((NC, BN, ED), lambda i: (0, i, 0)),
        pl.BlockSpec((D, D), lambda i: (0, 0)),
        pl.BlockSpec((1, D), lambda i: (0, 0)),
        pl.BlockSpec((D, D), lambda i: (0, 0)),
        pl.BlockSpec((D, D), lambda i: (0, 0)),
        pl.BlockSpec((1, D), lambda i: (0, 0)),
    ],
    out_specs=pl.BlockSpec((BN, D), lambda i: (i, 0)),
    out_shape=jax.ShapeDtypeStruct((N, D), jnp.float32),
)


def _final_body(h_ref, w_ref, b_ref, batch_ref, node_ref, graph_ref,
                sum_ref, cnt_ref):
    i = pl.program_id(0)
    ne = _mm(h_ref[...], w_ref[...]) + b_ref[...]
    node_ref[...] = ne
    bt = batch_ref[0, 0, :]
    gids = lax.broadcasted_iota(jnp.int32, (G, BN), 0)
    oh = (gids == bt[None, :]).astype(jnp.float32)

    @pl.when(i == 0)
    def _():
        sum_ref[...] = jnp.zeros((G, D), jnp.float32)
        cnt_ref[...] = jnp.zeros((G, D), jnp.float32)

    sum_ref[...] += _mm(oh, ne)
    cnt_ref[...] += jnp.broadcast_to(jnp.sum(oh, axis=1, keepdims=True), (G, D))

    @pl.when(i == GN - 1)
    def _():
        graph_ref[...] = sum_ref[...] / jnp.maximum(cnt_ref[...], 1.0)


_final_call = pl.pallas_call(
    _final_body,
    grid=(GN,),
    in_specs=[
        pl.BlockSpec((BN, D), lambda i: (i, 0)),
        pl.BlockSpec((D, D), lambda i: (0, 0)),
        pl.BlockSpec((1, D), lambda i: (0, 0)),
        pl.BlockSpec((1, 1, BN), lambda i: (i, 0, 0)),
    ],
    out_specs=[
        pl.BlockSpec((BN, D), lambda i: (i, 0)),
        pl.BlockSpec((G, D), lambda i: (0, 0)),
    ],
    out_shape=[
        jax.ShapeDtypeStruct((N, D), jnp.float32),
        jax.ShapeDtypeStruct((G, D), jnp.float32),
    ],
    scratch_shapes=[
        pltpu.VMEM((G, D), jnp.float32),
        pltpu.VMEM((G, D), jnp.float32),
    ],
)


_sc_mesh = plsc.VectorSubcoreMesh(core_axis_name="c", subcore_axis_name="s",
                                  num_cores=NC, num_subcores=NS)


@functools.partial(
    pl.kernel,
    out_type=jax.ShapeDtypeStruct((NC, NP, D), jnp.float32),
    mesh=_sc_mesh,
    scratch_types=[
        pltpu.VMEM((CH,), jnp.int32),
        pltpu.VMEM((CH,), jnp.int32),
        pltpu.VMEM((CH, D), jnp.float32),
        pltpu.VMEM((CH, D), jnp.float32),
        pltpu.VMEM((CH, D), jnp.float32),
        pltpu.VMEM((RZ, D), jnp.float32),
        pltpu.VMEM_SHARED((NP, D), jnp.float32),
        pltpu.SemaphoreType.DMA,
        pltpu.SemaphoreType.DMA,
        pltpu.SemaphoreType.DMA,
    ],
)
def _sc_agg(a_hbm, b_hbm, c_hbm, src_hbm, dst_hbm, out_hbm,
            src_v, dst_v, a_v, b_v, c_v, z_v, s_sh, sem_a, sem_b, sem_c):
    cid = lax.axis_index("c")
    sid = lax.axis_index("s")
    wid = cid * NS + sid

    def _zrow(r, carry):
        for kk in range(D // 16):
            z_v[r, pl.ds(kk * 16, 16)] = jnp.zeros((16,), jnp.float32)
        return carry

    lax.fori_loop(0, RZ, _zrow, 0)
    row0 = sid * RPT
    for k in range(RPT // RZ):
        pltpu.sync_copy(z_v, s_sh.at[pl.ds(row0 + k * RZ, RZ)])
    plsc.subcore_barrier()

    ebase = wid * EW

    def _chunk(j, carry):
        base = ebase + j * CH
        pltpu.sync_copy(src_hbm.at[pl.ds(base, CH)], src_v)
        pltpu.sync_copy(dst_hbm.at[pl.ds(base, CH)], dst_v)
        ca = pltpu.async_copy(a_hbm.at[src_v], a_v, sem_a)
        cb = pltpu.async_copy(b_hbm.at[dst_v], b_v, sem_b)
        cc = pltpu.async_copy(c_hbm.at[pl.ds(base, CH)], c_v, sem_c)
        ca.wait()
        cb.wait()
        cc.wait()

        def _crow(r, inner):
            for kk in range(D // 16):
                sl = pl.ds(kk * 16, 16)
                a_v[r, sl] = jnp.maximum(a_v[r, sl] + b_v[r, sl] + c_v[r, sl],
                                         0.0)
            return inner

        lax.fori_loop(0, CH, _crow, 0)
        pltpu.sync_copy(a_v, s_sh.at[dst_v], add=True)
        return carry

    lax.fori_loop(0, NCHUNK, _chunk, 0)

    plsc.subcore_barrier()
    pltpu.sync_copy(s_sh.at[pl.ds(row0, RPT)],
                    out_hbm.at[cid, pl.ds(row0, RPT)])


@functools.partial(
    pl.kernel,
    out_type=jax.ShapeDtypeStruct((NC, NP, ED), jnp.float32),
    mesh=_sc_mesh,
    scratch_types=[
        pltpu.VMEM((CH,), jnp.int32),
        pltpu.VMEM((CH, ED), jnp.float32),
        pltpu.VMEM((RZ, ED), jnp.float32),
        pltpu.VMEM_SHARED((NP, ED), jnp.float32),
    ],
)
def _sc_deg(dst_hbm, out_hbm, dst_v, one_v, z_v, s_sh):
    cid = lax.axis_index("c")
    sid = lax.axis_index("s")
    wid = cid * NS + sid

    def _fill(r, carry):
        z_v[r, pl.ds(0, 16)] = jnp.zeros((16,), jnp.float32)
        return carry

    lax.fori_loop(0, RZ, _fill, 0)

    def _fill1(r, carry):
        one_v[r, pl.ds(0, 16)] = jnp.ones((16,), jnp.float32)
        return carry

    lax.fori_loop(0, CH, _fill1, 0)

    row0 = sid * RPT
    for k in range(RPT // RZ):
        pltpu.sync_copy(z_v, s_sh.at[pl.ds(row0 + k * RZ, RZ)])
    plsc.subcore_barrier()

    ebase = wid * EW

    def _chunk(j, carry):
        base = ebase + j * CH
        pltpu.sync_copy(dst_hbm.at[pl.ds(base, CH)], dst_v)
        pltpu.sync_copy(one_v, s_sh.at[dst_v], add=True)
        return carry

    lax.fori_loop(0, NCHUNK, _chunk, 0)

    plsc.subcore_barrier()
    pltpu.sync_copy(s_sh.at[pl.ds(row0, RPT)],
                    out_hbm.at[cid, pl.ds(row0, RPT)])


def kernel(x, edge_index, edge_attr, pos, batch, params):
    del pos
    src = edge_index[0]
    dst = edge_index[1]

    eW, eb = params['emb']
    h = _emb_call(x, eW, eb.reshape(1, D))
    deg = _sc_deg(dst)

    for lp in params['layers']:
        nW1, nb1, nW2, nb2 = lp['node']
        e1W, e1b, e2W, e2b = lp['edge']
        uW, ub = lp['upd']
        hn, a, b = _stage1_call(h, nW1, nb1.reshape(1, D), nW2,
                                nb2.reshape(1, D), e1W[:D], e1W[D:2 * D])
        c = _cmat_call(edge_attr, e1W[2 * D:], e1b.reshape(1, D))
        s = _sc_agg(a, b, c, src, dst)
        h = _stage2_call(hn, s, deg, e2W, e2b.reshape(1, D), uW[:D], uW[D:],
                         ub.reshape(1, D))

    oW, ob = params['out']
    node_embeddings, graph_embedding = _final_call(
        h, oW, ob.reshape(1, D), batch.reshape(GN, 1, BN))
    return node_embeddings, graph_embedding


# speedup vs baseline: 3.9489x; 1.3317x over previous
import functools

import jax
import jax.numpy as jnp
from jax import lax
from jax.experimental import pallas as pl
from jax.experimental.pallas import tpu as pltpu
from jax.experimental.pallas import tpu_sc as plsc

N = 10000
E = 320000
D = 128
ED = 16
G = 16

NC = 2
NS = 16
NW = NC * NS
EW = E // NW
CH = 80
NCHUNK = EW // CH
NP = 10112
RPT = NP // NS
RZ = 128
CHA = 40
NCHA = EW // CHA
GRP = 10
NG = NCHA // GRP

BN = 1000
GN = N // BN
BE = 4000
GE = E // BE


def _mm(a, w):
    return lax.dot_general(a, w, (((1,), (0,)), ((), ())),
                           preferred_element_type=jnp.float32,
                           precision=lax.Precision.HIGHEST)


def _emb_body(x_ref, w_ref, b_ref, o_ref):
    o_ref[...] = _mm(x_ref[...], w_ref[...]) + b_ref[...]


_emb_call = pl.pallas_call(
    _emb_body,
    grid=(GN,),
    in_specs=[
        pl.BlockSpec((BN, D), lambda i: (i, 0)),
        pl.BlockSpec((D, D), lambda i: (0, 0)),
        pl.BlockSpec((1, D), lambda i: (0, 0)),
    ],
    out_specs=pl.BlockSpec((BN, D), lambda i: (i, 0)),
    out_shape=jax.ShapeDtypeStruct((N, D), jnp.float32),
)


def _stage1_body(h_ref, w1_ref, b1_ref, w2_ref, b2_ref, ea_ref, eb_ref,
                 hn_ref, a_ref, b_ref):
    h = h_ref[...]
    p = jnp.maximum(_mm(h, w1_ref[...]) + b1_ref[...], 0.0)
    hn_ref[...] = _mm(p, w2_ref[...]) + b2_ref[...]
    a_ref[...] = _mm(h, ea_ref[...])
    b_ref[...] = _mm(h, eb_ref[...])


_stage1_call = pl.pallas_call(
    _stage1_body,
    grid=(GN,),
    in_specs=[
        pl.BlockSpec((BN, D), lambda i: (i, 0)),
        pl.BlockSpec((D, D), lambda i: (0, 0)),
        pl.BlockSpec((1, D), lambda i: (0, 0)),
        pl.BlockSpec((D, D), lambda i: (0, 0)),
        pl.BlockSpec((1, D), lambda i: (0, 0)),
        pl.BlockSpec((D, D), lambda i: (0, 0)),
        pl.BlockSpec((D, D), lambda i: (0, 0)),
    ],
    out_specs=[pl.BlockSpec((BN, D), lambda i: (i, 0))] * 3,
    out_shape=[jax.ShapeDtypeStruct((N, D), jnp.float32)] * 3,
)


def _cmat_body(ea_ref, w_ref, b_ref, o_ref):
    o_ref[...] = _mm(ea_ref[...], w_ref[...]) + b_ref[...]


_cmat_call = pl.pallas_call(
    _cmat_body,
    grid=(GE,),
    in_specs=[
        pl.BlockSpec((BE, ED), lambda i: (i, 0)),
        pl.BlockSpec((ED, D), lambda i: (0, 0)),
        pl.BlockSpec((1, D), lambda i: (0, 0)),
    ],
    out_specs=pl.BlockSpec((BE, D), lambda i: (i, 0)),
    out_shape=jax.ShapeDtypeStruct((E, D), jnp.float32),
)


def _stage2_body(hn_ref, s_ref, deg_ref, e2w_ref, e2b_ref, uwa_ref, uwb_ref,
                 ub_ref, o_ref):
    s = s_ref[0] + s_ref[1]
    degc = deg_ref[0, :, 0:1] + deg_ref[1, :, 0:1]
    hagg = _mm(s, e2w_ref[...]) + degc * e2b_ref[...]
    o_ref[...] = _mm(hn_ref[...], uwa_ref[...]) + _mm(hagg, uwb_ref[...]) + ub_ref[...]


_stage2_call = pl.pallas_call(
    _stage2_body,
    grid=(GN,),
    in_specs=[
        pl.BlockSpec((BN, D), lambda i: (i, 0)),
        pl.BlockSpec((NC, BN, D), lambda i: (0, i, 0)),
        pl.BlockSpec((NC, BN, ED), lambda i: (0, i, 0)),
        pl.BlockSpec((D, D), lambda i: (0, 0)),
        pl.BlockSpec((1, D), lambda i: (0, 0)),
        pl.BlockSpec((D, D), lambda i: (0, 0)),
        pl.BlockSpec((D, D), lambda i: (0, 0)),
        pl.BlockSpec((1, D), lambda i: (0, 0)),
    ],
    out_specs=pl.BlockSpec((BN, D), lambda i: (i, 0)),
    out_shape=jax.ShapeDtypeStruct((N, D), jnp.float32),
)


def _final_body(h_ref, w_ref, b_ref, batch_ref, node_ref, graph_ref,
                sum_ref, cnt_ref):
    i = pl.program_id(0)
    ne = _mm(h_ref[...], w_ref[...]) + b_ref[...]
    node_ref[...] = ne
    bt = batch_ref[0, 0, :]
    gids = lax.broadcasted_iota(jnp.int32, (G, BN), 0)
    oh = (gids == bt[None, :]).astype(jnp.float32)

    @pl.when(i == 0)
    def _():
        sum_ref[...] = jnp.zeros((G, D), jnp.float32)
        cnt_ref[...] = jnp.zeros((G, D), jnp.float32)

    sum_ref[...] += _mm(oh, ne)
    cnt_ref[...] += jnp.broadcast_to(jnp.sum(oh, axis=1, keepdims=True), (G, D))

    @pl.when(i == GN - 1)
    def _():
        graph_ref[...] = sum_ref[...] / jnp.maximum(cnt_ref[...], 1.0)


_final_call = pl.pallas_call(
    _final_body,
    grid=(GN,),
    in_specs=[
        pl.BlockSpec((BN, D), lambda i: (i, 0)),
        pl.BlockSpec((D, D), lambda i: (0, 0)),
        pl.BlockSpec((1, D), lambda i: (0, 0)),
        pl.BlockSpec((1, 1, BN), lambda i: (i, 0, 0)),
    ],
    out_specs=[
        pl.BlockSpec((BN, D), lambda i: (i, 0)),
        pl.BlockSpec((G, D), lambda i: (0, 0)),
    ],
    out_shape=[
        jax.ShapeDtypeStruct((N, D), jnp.float32),
        jax.ShapeDtypeStruct((G, D), jnp.float32),
    ],
    scratch_shapes=[
        pltpu.VMEM((G, D), jnp.float32),
        pltpu.VMEM((G, D), jnp.float32),
    ],
)


_sc_mesh = plsc.VectorSubcoreMesh(core_axis_name="c", subcore_axis_name="s",
                                  num_cores=NC, num_subcores=NS)


@functools.partial(
    pl.kernel,
    out_type=jax.ShapeDtypeStruct((NC, NP, D), jnp.float32),
    mesh=_sc_mesh,
    scratch_types=[
        pltpu.VMEM((GRP, CHA), jnp.int32),
        pltpu.VMEM((GRP, CHA), jnp.int32),
        pltpu.VMEM((CHA, D), jnp.float32),
        pltpu.VMEM((CHA, D), jnp.float32),
        pltpu.VMEM((CHA, D), jnp.float32),
        pltpu.VMEM((CHA, D), jnp.float32),
        pltpu.VMEM((CHA, D), jnp.float32),
        pltpu.VMEM((CHA, D), jnp.float32),
        pltpu.VMEM((CHA, D), jnp.float32),
        pltpu.VMEM((CHA, D), jnp.float32),
        pltpu.VMEM_SHARED((NP, D), jnp.float32),
        pltpu.SemaphoreType.DMA,
        pltpu.SemaphoreType.DMA,
        pltpu.SemaphoreType.DMA,
        pltpu.SemaphoreType.DMA,
    ],
)
def _sc_agg(a_hbm, b_hbm, c_hbm, src_hbm, dst_hbm, out_hbm,
            sbig, dbig, a0, b0, c0, o0, a1, b1, c1, o1, s_sh,
            sem_i0, sem_i1, sem_s0, sem_s1):
    cid = lax.axis_index("c")
    sid = lax.axis_index("s")
    wid = cid * NS + sid
    ebase = wid * EW
    bufs = ((a0, b0, c0, o0, sem_i0, sem_s0),
            (a1, b1, c1, o1, sem_i1, sem_s1))

    def _zrow(r, carry):
        for kk in range(D // 16):
            o0[r, pl.ds(kk * 16, 16)] = jnp.zeros((16,), jnp.float32)
        return carry

    lax.fori_loop(0, CHA, _zrow, 0)
    row0 = sid * RPT
    for k in range(RPT // CHA):
        pltpu.sync_copy(o0, s_sh.at[pl.ds(row0 + k * CHA, CHA)])
    _ztail = RPT % CHA
    if _ztail:
        pltpu.sync_copy(o0.at[pl.ds(0, _ztail)],
                        s_sh.at[pl.ds(row0 + RPT - _ztail, _ztail)])
    plsc.subcore_barrier()

    def _wait_in(a_v, b_v, c_v, sem):
        pltpu.make_async_copy(a_hbm.at[pl.ds(0, CHA)], a_v, sem).wait()
        pltpu.make_async_copy(b_hbm.at[pl.ds(0, CHA)], b_v, sem).wait()
        pltpu.make_async_copy(c_hbm.at[pl.ds(0, CHA)], c_v, sem).wait()

    def _wait_sc(o_v, sem):
        pltpu.make_async_copy(o_v, s_sh.at[pl.ds(0, CHA)], sem).wait()

    def _compute(a_v, b_v, c_v, o_v):
        def _crow(r, inner):
            for kk in range(D // 16):
                sl = pl.ds(kk * 16, 16)
                o_v[r, sl] = jnp.maximum(a_v[r, sl] + b_v[r, sl] + c_v[r, sl],
                                         0.0)
            return inner

        lax.fori_loop(0, CHA, _crow, 0)

    for g in range(NG):
        gb = ebase + g * GRP * CHA

        def _fire_in(k, a_v, b_v, c_v, sem, _gb=gb):
            pltpu.async_copy(a_hbm.at[sbig.at[k]], a_v, sem)
            pltpu.async_copy(b_hbm.at[dbig.at[k]], b_v, sem)
            pltpu.async_copy(c_hbm.at[pl.ds(_gb + k * CHA, CHA)], c_v, sem)

        if g > 0:
            _wait_sc(o0, sem_s0)
            _wait_sc(o1, sem_s1)
        pltpu.sync_copy(src_hbm.at[wid, g, pl.ds(0, GRP)], sbig)
        pltpu.sync_copy(dst_hbm.at[wid, g, pl.ds(0, GRP)], dbig)
        _fire_in(0, a0, b0, c0, sem_i0)
        _fire_in(1, a1, b1, c1, sem_i1)

        def _pair(it, carry):
            k0 = 2 * it
            for p in range(2):
                a_v, b_v, c_v, o_v, sem_i, sem_s = bufs[p]
                k = k0 + p
                _wait_in(a_v, b_v, c_v, sem_i)

                @pl.when(k >= 2)
                def _():
                    _wait_sc(o_v, sem_s)

                _compute(a_v, b_v, c_v, o_v)

                @pl.when(k + 2 < GRP)
                def _():
                    _fire_in(k + 2, a_v, b_v, c_v, sem_i)

                pltpu.async_copy(o_v, s_sh.at[dbig.at[k]], sem_s, add=True)
            return carry

        lax.fori_loop(0, GRP // 2, _pair, 0)

    _wait_sc(o0, sem_s0)
    _wait_sc(o1, sem_s1)

    plsc.subcore_barrier()
    pltpu.sync_copy(s_sh.at[pl.ds(row0, RPT)],
                    out_hbm.at[cid, pl.ds(row0, RPT)])


@functools.partial(
    pl.kernel,
    out_type=jax.ShapeDtypeStruct((NC, NP, ED), jnp.float32),
    mesh=_sc_mesh,
    scratch_types=[
        pltpu.VMEM((CH,), jnp.int32),
        pltpu.VMEM((CH, ED), jnp.float32),
        pltpu.VMEM((RZ, ED), jnp.float32),
        pltpu.VMEM_SHARED((NP, ED), jnp.float32),
    ],
)
def _sc_deg(dst_hbm, out_hbm, dst_v, one_v, z_v, s_sh):
    cid = lax.axis_index("c")
    sid = lax.axis_index("s")
    wid = cid * NS + sid

    def _fill(r, carry):
        z_v[r, pl.ds(0, 16)] = jnp.zeros((16,), jnp.float32)
        return carry

    lax.fori_loop(0, RZ, _fill, 0)

    def _fill1(r, carry):
        one_v[r, pl.ds(0, 16)] = jnp.ones((16,), jnp.float32)
        return carry

    lax.fori_loop(0, CH, _fill1, 0)

    row0 = sid * RPT
    for k in range(RPT // RZ):
        pltpu.sync_copy(z_v, s_sh.at[pl.ds(row0 + k * RZ, RZ)])
    plsc.subcore_barrier()

    ebase = wid * EW

    def _chunk(j, carry):
        base = ebase + j * CH
        pltpu.sync_copy(dst_hbm.at[pl.ds(base, CH)], dst_v)
        pltpu.sync_copy(one_v, s_sh.at[dst_v], add=True)
        return carry

    lax.fori_loop(0, NCHUNK, _chunk, 0)

    plsc.subcore_barrier()
    pltpu.sync_copy(s_sh.at[pl.ds(row0, RPT)],
                    out_hbm.at[cid, pl.ds(row0, RPT)])


def kernel(x, edge_index, edge_attr, pos, batch, params):
    del pos
    src = edge_index[0]
    dst = edge_index[1]
    src3 = src.reshape(NW, NG, GRP, CHA)
    dst3 = dst.reshape(NW, NG, GRP, CHA)

    eW, eb = params['emb']
    h = _emb_call(x, eW, eb.reshape(1, D))
    deg = _sc_deg(dst)

    for lp in params['layers']:
        nW1, nb1, nW2, nb2 = lp['node']
        e1W, e1b, e2W, e2b = lp['edge']
        uW, ub = lp['upd']
        hn, a, b = _stage1_call(h, nW1, nb1.reshape(1, D), nW2,
                                nb2.reshape(1, D), e1W[:D], e1W[D:2 * D])
        c = _cmat_call(edge_attr, e1W[2 * D:], e1b.reshape(1, D))
        s = _sc_agg(a, b, c, src3, dst3)
        h = _stage2_call(hn, s, deg, e2W, e2b.reshape(1, D), uW[:D], uW[D:],
                         ub.reshape(1, D))

    oW, ob = params['out']
    node_embeddings, graph_embedding = _final_call(
        h, oW, ob.reshape(1, D), batch.reshape(GN, 1, BN))
    return node_embeddings, graph_embedding


# speedup vs baseline: 4.1788x; 1.0582x over previous
import functools

import jax
import jax.numpy as jnp
from jax import lax
from jax.experimental import pallas as pl
from jax.experimental.pallas import tpu as pltpu
from jax.experimental.pallas import tpu_sc as plsc

N = 10000
E = 320000
D = 128
ED = 16
G = 16

NC = 2
NS = 16
NW = NC * NS
EW = E // NW
CH = 80
NCHUNK = EW // CH
NP = 10112
RPT = NP // NS
RZ = 128
CHA = 40
NCHA = EW // CHA
GRP = 10
NG = NCHA // GRP

BN = 1000
GN = N // BN
BE = 4000
GE = E // BE


def _mm(a, w):
    return lax.dot_general(a, w, (((1,), (0,)), ((), ())),
                           preferred_element_type=jnp.float32,
                           precision=lax.Precision.HIGHEST)


def _emb_body(x_ref, w_ref, b_ref, o_ref):
    o_ref[...] = _mm(x_ref[...], w_ref[...]) + b_ref[...]


_emb_call = pl.pallas_call(
    _emb_body,
    grid=(GN,),
    in_specs=[
        pl.BlockSpec((BN, D), lambda i: (i, 0)),
        pl.BlockSpec((D, D), lambda i: (0, 0)),
        pl.BlockSpec((1, D), lambda i: (0, 0)),
    ],
    out_specs=pl.BlockSpec((BN, D), lambda i: (i, 0)),
    out_shape=jax.ShapeDtypeStruct((N, D), jnp.float32),
)


def _stage1_body(h_ref, w1_ref, b1_ref, w2_ref, b2_ref, ea_ref, eb_ref,
                 hn_ref, a_ref, b_ref):
    h = h_ref[...]
    p = jnp.maximum(_mm(h, w1_ref[...]) + b1_ref[...], 0.0)
    hn_ref[...] = _mm(p, w2_ref[...]) + b2_ref[...]
    a_ref[...] = _mm(h, ea_ref[...])
    b_ref[...] = _mm(h, eb_ref[...])


_stage1_call = pl.pallas_call(
    _stage1_body,
    grid=(GN,),
    in_specs=[
        pl.BlockSpec((BN, D), lambda i: (i, 0)),
        pl.BlockSpec((D, D), lambda i: (0, 0)),
        pl.BlockSpec((1, D), lambda i: (0, 0)),
        pl.BlockSpec((D, D), lambda i: (0, 0)),
        pl.BlockSpec((1, D), lambda i: (0, 0)),
        pl.BlockSpec((D, D), lambda i: (0, 0)),
        pl.BlockSpec((D, D), lambda i: (0, 0)),
    ],
    out_specs=[pl.BlockSpec((BN, D), lambda i: (i, 0))] * 3,
    out_shape=[jax.ShapeDtypeStruct((N, D), jnp.float32)] * 3,
)


def _cmat_body(ea_ref, w_ref, b_ref, o_ref):
    o_ref[...] = _mm(ea_ref[...], w_ref[...]) + b_ref[...]


_cmat_call = pl.pallas_call(
    _cmat_body,
    grid=(GE,),
    in_specs=[
        pl.BlockSpec((BE, ED), lambda i: (i, 0)),
        pl.BlockSpec((ED, D), lambda i: (0, 0)),
        pl.BlockSpec((1, D), lambda i: (0, 0)),
    ],
    out_specs=pl.BlockSpec((BE, D), lambda i: (i, 0)),
    out_shape=jax.ShapeDtypeStruct((E, D), jnp.float32),
)


def _stage2_body(hn_ref, s_ref, deg_ref, e2w_ref, e2b_ref, uwa_ref, uwb_ref,
                 ub_ref, o_ref):
    s = s_ref[0] + s_ref[1]
    degc = deg_ref[0, :, 0:1] + deg_ref[1, :, 0:1]
    hagg = _mm(s, e2w_ref[...]) + degc * e2b_ref[...]
    o_ref[...] = _mm(hn_ref[...], uwa_ref[...]) + _mm(hagg, uwb_ref[...]) + ub_ref[...]


_stage2_call = pl.pallas_call(
    _stage2_body,
    grid=(GN,),
    in_specs=[
        pl.BlockSpec((BN, D), lambda i: (i, 0)),
        pl.BlockSpec((NC, BN, D), lambda i: (0, i, 0)),
        pl.BlockSpec((NC, BN, ED), lambda i: (0, i, 0)),
        pl.BlockSpec((D, D), lambda i: (0, 0)),
        pl.BlockSpec((1, D), lambda i: (0, 0)),
        pl.BlockSpec((D, D), lambda i: (0, 0)),
        pl.BlockSpec((D, D), lambda i: (0, 0)),
        pl.BlockSpec((1, D), lambda i: (0, 0)),
    ],
    out_specs=pl.BlockSpec((BN, D), lambda i: (i, 0)),
    out_shape=jax.ShapeDtypeStruct((N, D), jnp.float32),
)


def _final_body(h_ref, w_ref, b_ref, batch_ref, node_ref, graph_ref,
                sum_ref, cnt_ref):
    i = pl.program_id(0)
    ne = _mm(h_ref[...], w_ref[...]) + b_ref[...]
    node_ref[...] = ne
    bt = batch_ref[0, 0, :]
    gids = lax.broadcasted_iota(jnp.int32, (G, BN), 0)
    oh = (gids == bt[None, :]).astype(jnp.float32)

    @pl.when(i == 0)
    def _():
        sum_ref[...] = jnp.zeros((G, D), jnp.float32)
        cnt_ref[...] = jnp.zeros((G, D), jnp.float32)

    sum_ref[...] += _mm(oh, ne)
    cnt_ref[...] += jnp.broadcast_to(jnp.sum(oh, axis=1, keepdims=True), (G, D))

    @pl.when(i == GN - 1)
    def _():
        graph_ref[...] = sum_ref[...] / jnp.maximum(cnt_ref[...], 1.0)


_final_call = pl.pallas_call(
    _final_body,
    grid=(GN,),
    in_specs=[
        pl.BlockSpec((BN, D), lambda i: (i, 0)),
        pl.BlockSpec((D, D), lambda i: (0, 0)),
        pl.BlockSpec((1, D), lambda i: (0, 0)),
        pl.BlockSpec((1, 1, BN), lambda i: (i, 0, 0)),
    ],
    out_specs=[
        pl.BlockSpec((BN, D), lambda i: (i, 0)),
        pl.BlockSpec((G, D), lambda i: (0, 0)),
    ],
    out_shape=[
        jax.ShapeDtypeStruct((N, D), jnp.float32),
        jax.ShapeDtypeStruct((G, D), jnp.float32),
    ],
    scratch_shapes=[
        pltpu.VMEM((G, D), jnp.float32),
        pltpu.VMEM((G, D), jnp.float32),
    ],
)


_sc_mesh = plsc.VectorSubcoreMesh(core_axis_name="c", subcore_axis_name="s",
                                  num_cores=NC, num_subcores=NS)


@functools.partial(
    pl.kernel,
    out_type=jax.ShapeDtypeStruct((NC, NP, D), jnp.float32),
    mesh=_sc_mesh,
    scratch_types=[
        pltpu.VMEM((GRP, CHA), jnp.int32),
        pltpu.VMEM((GRP, CHA), jnp.int32),
        pltpu.VMEM((GRP, CHA), jnp.int32),
        pltpu.VMEM((GRP, CHA), jnp.int32),
        pltpu.VMEM((CHA, D), jnp.float32),
        pltpu.VMEM((CHA, D), jnp.float32),
        pltpu.VMEM((CHA, D), jnp.float32),
        pltpu.VMEM((CHA, D), jnp.float32),
        pltpu.VMEM((CHA, D), jnp.float32),
        pltpu.VMEM((CHA, D), jnp.float32),
        pltpu.VMEM((CHA, D), jnp.float32),
        pltpu.VMEM((CHA, D), jnp.float32),
        pltpu.VMEM_SHARED((NP, D), jnp.float32),
        pltpu.SemaphoreType.DMA,
        pltpu.SemaphoreType.DMA,
        pltpu.SemaphoreType.DMA,
        pltpu.SemaphoreType.DMA,
        pltpu.SemaphoreType.DMA,
    ],
)
def _sc_agg(a_hbm, b_hbm, c_hbm, src_hbm, dst_hbm, out_hbm,
            sbig0, dbig0, sbig1, dbig1, a0, b0, c0, o0, a1, b1, c1, o1, s_sh,
            sem_i0, sem_i1, sem_s0, sem_s1, sem_g):
    cid = lax.axis_index("c")
    sid = lax.axis_index("s")
    wid = cid * NS + sid
    ebase = wid * EW
    bufs = ((a0, b0, c0, o0, sem_i0, sem_s0),
            (a1, b1, c1, o1, sem_i1, sem_s1))

    def _zrow(r, carry):
        for kk in range(D // 16):
            o0[r, pl.ds(kk * 16, 16)] = jnp.zeros((16,), jnp.float32)
        return carry

    lax.fori_loop(0, CHA, _zrow, 0)
    row0 = sid * RPT
    for k in range(RPT // CHA):
        pltpu.sync_copy(o0, s_sh.at[pl.ds(row0 + k * CHA, CHA)])
    _ztail = RPT % CHA
    if _ztail:
        pltpu.sync_copy(o0.at[pl.ds(0, _ztail)],
                        s_sh.at[pl.ds(row0 + RPT - _ztail, _ztail)])
    plsc.subcore_barrier()

    def _wait_in(a_v, b_v, c_v, sem):
        pltpu.make_async_copy(a_hbm.at[pl.ds(0, CHA)], a_v, sem).wait()
        pltpu.make_async_copy(b_hbm.at[pl.ds(0, CHA)], b_v, sem).wait()
        pltpu.make_async_copy(c_hbm.at[pl.ds(0, CHA)], c_v, sem).wait()

    def _wait_sc(o_v, sem):
        pltpu.make_async_copy(o_v, s_sh.at[pl.ds(0, CHA)], sem).wait()

    def _compute(a_v, b_v, c_v, o_v):
        def _crow(r, inner):
            for kk in range(D // 16):
                sl = pl.ds(kk * 16, 16)
                o_v[r, sl] = jnp.maximum(a_v[r, sl] + b_v[r, sl] + c_v[r, sl],
                                         0.0)
            return inner

        lax.fori_loop(0, CHA, _crow, 0)

    def _fire_idx(g, sb, db):
        pltpu.async_copy(src_hbm.at[wid, g, pl.ds(0, GRP)], sb, sem_g)
        pltpu.async_copy(dst_hbm.at[wid, g, pl.ds(0, GRP)], db, sem_g)

    def _wait_idx(sb, db):
        pltpu.make_async_copy(src_hbm.at[0, 0, pl.ds(0, GRP)], sb,
                              sem_g).wait()
        pltpu.make_async_copy(dst_hbm.at[0, 0, pl.ds(0, GRP)], db,
                              sem_g).wait()

    _fire_idx(0, sbig0, dbig0)
    for g in range(NG):
        sbig, dbig = (sbig0, dbig0) if g % 2 == 0 else (sbig1, dbig1)
        sbn, dbn = (sbig1, dbig1) if g % 2 == 0 else (sbig0, dbig0)
        gb = ebase + g * GRP * CHA

        def _fire_in(k, a_v, b_v, c_v, sem, _gb=gb, _sb=sbig, _db=dbig):
            pltpu.async_copy(a_hbm.at[_sb.at[k]], a_v, sem)
            pltpu.async_copy(b_hbm.at[_db.at[k]], b_v, sem)
            pltpu.async_copy(c_hbm.at[pl.ds(_gb + k * CHA, CHA)], c_v, sem)

        _wait_idx(sbig, dbig)
        _fire_in(0, a0, b0, c0, sem_i0)
        _fire_in(1, a1, b1, c1, sem_i1)
        if g > 0:
            _wait_sc(o0, sem_s0)
            _wait_sc(o1, sem_s1)
        if g + 1 < NG:
            _fire_idx(g + 1, sbn, dbn)

        def _pair(it, carry):
            k0 = 2 * it
            for p in range(2):
                a_v, b_v, c_v, o_v, sem_i, sem_s = bufs[p]
                k = k0 + p
                _wait_in(a_v, b_v, c_v, sem_i)

                @pl.when(k >= 2)
                def _():
                    _wait_sc(o_v, sem_s)

                _compute(a_v, b_v, c_v, o_v)

                @pl.when(k + 2 < GRP)
                def _():
                    _fire_in(k + 2, a_v, b_v, c_v, sem_i)

                pltpu.async_copy(o_v, s_sh.at[dbig.at[k]], sem_s, add=True)
            return carry

        lax.fori_loop(0, GRP // 2, _pair, 0)

    _wait_sc(o0, sem_s0)
    _wait_sc(o1, sem_s1)

    plsc.subcore_barrier()
    pltpu.sync_copy(s_sh.at[pl.ds(row0, RPT)],
                    out_hbm.at[cid, pl.ds(row0, RPT)])


@functools.partial(
    pl.kernel,
    out_type=jax.ShapeDtypeStruct((NC, NP, ED), jnp.float32),
    mesh=_sc_mesh,
    scratch_types=[
        pltpu.VMEM((CH,), jnp.int32),
        pltpu.VMEM((CH, ED), jnp.float32),
        pltpu.VMEM((RZ, ED), jnp.float32),
        pltpu.VMEM_SHARED((NP, ED), jnp.float32),
    ],
)
def _sc_deg(dst_hbm, out_hbm, dst_v, one_v, z_v, s_sh):
    cid = lax.axis_index("c")
    sid = lax.axis_index("s")
    wid = cid * NS + sid

    def _fill(r, carry):
        z_v[r, pl.ds(0, 16)] = jnp.zeros((16,), jnp.float32)
        return carry

    lax.fori_loop(0, RZ, _fill, 0)

    def _fill1(r, carry):
        one_v[r, pl.ds(0, 16)] = jnp.ones((16,), jnp.float32)
        return carry

    lax.fori_loop(0, CH, _fill1, 0)

    row0 = sid * RPT
    for k in range(RPT // RZ):
        pltpu.sync_copy(z_v, s_sh.at[pl.ds(row0 + k * RZ, RZ)])
    _dtail = RPT % RZ
    if _dtail:
        pltpu.sync_copy(z_v.at[pl.ds(0, _dtail)],
                        s_sh.at[pl.ds(row0 + RPT - _dtail, _dtail)])
    plsc.subcore_barrier()

    ebase = wid * EW

    def _chunk(j, carry):
        base = ebase + j * CH
        pltpu.sync_copy(dst_hbm.at[pl.ds(base, CH)], dst_v)
        pltpu.sync_copy(one_v, s_sh.at[dst_v], add=True)
        return carry

    lax.fori_loop(0, NCHUNK, _chunk, 0)

    plsc.subcore_barrier()
    pltpu.sync_copy(s_sh.at[pl.ds(row0, RPT)],
                    out_hbm.at[cid, pl.ds(row0, RPT)])


def kernel(x, edge_index, edge_attr, pos, batch, params):
    del pos
    src = edge_index[0]
    dst = edge_index[1]
    src3 = src.reshape(NW, NG, GRP, CHA)
    dst3 = dst.reshape(NW, NG, GRP, CHA)

    eW, eb = params['emb']
    h = _emb_call(x, eW, eb.reshape(1, D))
    deg = _sc_deg(dst)
    cs = [_cmat_call(edge_attr, lp['edge'][0][2 * D:],
                     lp['edge'][1].reshape(1, D)) for lp in params['layers']]

    for li, lp in enumerate(params['layers']):
        nW1, nb1, nW2, nb2 = lp['node']
        e1W, e1b, e2W, e2b = lp['edge']
        uW, ub = lp['upd']
        hn, a, b = _stage1_call(h, nW1, nb1.reshape(1, D), nW2,
                                nb2.reshape(1, D), e1W[:D], e1W[D:2 * D])
        s = _sc_agg(a, b, cs[li], src3, dst3)
        h = _stage2_call(hn, s, deg, e2W, e2b.reshape(1, D), uW[:D], uW[D:],
                         ub.reshape(1, D))

    oW, ob = params['out']
    node_embeddings, graph_embedding = _final_call(
        h, oW, ob.reshape(1, D), batch.reshape(GN, 1, BN))
    return node_embeddings, graph_embedding
